# speedup vs baseline: 1.0047x; 1.0047x over previous
import functools
import jax
import jax.numpy as jnp
from jax.experimental import pallas as pl
from jax.experimental.pallas import tpu as pltpu

N = 100000
E = 1600000
G = 256
C = 64
RB = 2000
GRID = N // RB


def _stats_body(y_ref, out_ref):
    i = pl.program_id(0)

    @pl.when(i == 0)
    def _():
        out_ref[...] = jnp.zeros_like(out_ref)

    y = y_ref[...]
    s1 = jnp.sum(y, axis=0, keepdims=True)
    s2 = jnp.sum(y * y, axis=0, keepdims=True)
    out_ref[...] += jnp.concatenate([s1, s2], axis=0)


def _moments(y):
    s = pl.pallas_call(
        _stats_body,
        grid=(GRID,),
        in_specs=[pl.BlockSpec((RB, C), lambda i: (i, 0))],
        out_specs=pl.BlockSpec((2, C), lambda i: (0, 0)),
        out_shape=jax.ShapeDtypeStruct((2, C), jnp.float32),
    )(y)
    mu = s[0] / N
    var = s[1] / N - mu * mu
    rstd = 1.0 / jnp.sqrt(var + 1e-5)
    return mu.reshape(1, C), rstd.reshape(1, C)


def _lin0_body(x_ref, w_ref, b_ref, y_ref):
    y_ref[...] = jnp.dot(x_ref[...], w_ref[...],
                         preferred_element_type=jnp.float32) + b_ref[...]


def _lin0(x, W0, b0):
    return pl.pallas_call(
        _lin0_body,
        grid=(GRID,),
        in_specs=[
            pl.BlockSpec((RB, 12), lambda i: (i, 0)),
            pl.BlockSpec((12, C), lambda i: (0, 0)),
            pl.BlockSpec((1, C), lambda i: (0, 0)),
        ],
        out_specs=pl.BlockSpec((RB, C), lambda i: (i, 0)),
        out_shape=jax.ShapeDtypeStruct((N, C), jnp.float32),
    )(x, W0, b0.reshape(1, C))


def _update_body(use_relu, has_res, y_ref, res_ref, mu_ref, rstd_ref, g_ref,
                 be_ref, A_ref, Ab_ref, B_ref, h_ref, p_ref, q_ref):
    h = (y_ref[...] - mu_ref[...]) * (g_ref[...] * rstd_ref[...]) + be_ref[...]
    if has_res:
        h = h + res_ref[...]
    if use_relu:
        h = jnp.maximum(h, 0.0)
    h_ref[...] = h
    p_ref[...] = jnp.dot(h, A_ref[...],
                         preferred_element_type=jnp.float32) + Ab_ref[...]
    q_ref[...] = jnp.dot(h, B_ref[...], preferred_element_type=jnp.float32)


def _update(y, res, mu, rstd, g, be, A, Ab, B, use_relu):
    has_res = res is not None
    if not has_res:
        res = jnp.zeros((1, C), jnp.float32)
    body = functools.partial(_update_body, use_relu, has_res)
    res_spec = (pl.BlockSpec((RB, C), lambda i: (i, 0)) if has_res
                else pl.BlockSpec((1, C), lambda i: (0, 0)))
    return pl.pallas_call(
        body,
        grid=(GRID,),
        in_specs=[
            pl.BlockSpec((RB, C), lambda i: (i, 0)),
            res_spec,
            pl.BlockSpec((1, C), lambda i: (0, 0)),
            pl.BlockSpec((1, C), lambda i: (0, 0)),
            pl.BlockSpec((1, C), lambda i: (0, 0)),
            pl.BlockSpec((1, C), lambda i: (0, 0)),
            pl.BlockSpec((C, 2 * C), lambda i: (0, 0)),
            pl.BlockSpec((1, 2 * C), lambda i: (0, 0)),
            pl.BlockSpec((C, 2 * C), lambda i: (0, 0)),
        ],
        out_specs=[
            pl.BlockSpec((RB, C), lambda i: (i, 0)),
            pl.BlockSpec((RB, 2 * C), lambda i: (i, 0)),
            pl.BlockSpec((RB, 2 * C), lambda i: (i, 0)),
        ],
        out_shape=[
            jax.ShapeDtypeStruct((N, C), jnp.float32),
            jax.ShapeDtypeStruct((N, 2 * C), jnp.float32),
            jax.ShapeDtypeStruct((N, 2 * C), jnp.float32),
        ],
    )(y, res, mu, rstd, g, be, A, Ab, B)


def _pool_body(agg_ref, h2_ref, mu_ref, rstd_ref, g_ref, be_ref, batch_ref,
               fcW1_ref, fcb1_ref, fcW2_ref, fcb2_ref, out_ref,
               acc_ref, cnt_ref):
    i = pl.program_id(0)

    @pl.when(i == 0)
    def _():
        acc_ref[...] = jnp.zeros_like(acc_ref)
        cnt_ref[...] = jnp.zeros_like(cnt_ref)

    h3 = (agg_ref[...] - mu_ref[...]) * (g_ref[...] * rstd_ref[...]) \
        + be_ref[...] + h2_ref[...]
    oh = (batch_ref[...] ==
          jax.lax.broadcasted_iota(jnp.int32, (1, G), 1)).astype(jnp.float32)
    acc_ref[...] += jax.lax.dot_general(
        oh, h3, (((0,), (0,)), ((), ())), preferred_element_type=jnp.float32)
    cnt_ref[...] += jnp.sum(oh, axis=0, keepdims=True)

    @pl.when(i == GRID - 1)
    def _():
        pooled = acc_ref[...] / jnp.maximum(cnt_ref[...], 1.0).T
        t = jnp.maximum(
            jnp.dot(pooled, fcW1_ref[...], preferred_element_type=jnp.float32)
            + fcb1_ref[...], 0.0)
        out_ref[...] = jnp.dot(t, fcW2_ref[...],
                               preferred_element_type=jnp.float32) + fcb2_ref[...]


def _pool_head(agg3, h2, mu, rstd, g, be, batch, fcW1, fcb1, fcW2, fcb2):
    return pl.pallas_call(
        _pool_body,
        grid=(GRID,),
        in_specs=[
            pl.BlockSpec((RB, C), lambda i: (i, 0)),
            pl.BlockSpec((RB, C), lambda i: (i, 0)),
            pl.BlockSpec((1, C), lambda i: (0, 0)),
            pl.BlockSpec((1, C), lambda i: (0, 0)),
            pl.BlockSpec((1, C), lambda i: (0, 0)),
            pl.BlockSpec((1, C), lambda i: (0, 0)),
            pl.BlockSpec((RB, 1), lambda i: (i, 0)),
            pl.BlockSpec((C, 32), lambda i: (0, 0)),
            pl.BlockSpec((1, 32), lambda i: (0, 0)),
            pl.BlockSpec((32, 1), lambda i: (0, 0)),
            pl.BlockSpec((1, 1), lambda i: (0, 0)),
        ],
        out_specs=pl.BlockSpec((G, 1), lambda i: (0, 0)),
        out_shape=jax.ShapeDtypeStruct((G, 1), jnp.float32),
        scratch_shapes=[pltpu.VMEM((G, C), jnp.float32),
                        pltpu.VMEM((1, G), jnp.float32)],
    )(agg3, h2, mu, rstd, g, be, batch.reshape(N, 1), fcW1,
      fcb1.reshape(1, 32), fcW2, fcb2.reshape(1, 1))


def _edge_pass(P, Q, we, src, dst, e):
    Z = P[dst] + Q[src] + e[:, None] * we
    u = Z[:, :C]
    v = Z[:, C:]
    msg = jax.nn.sigmoid(u) * jax.nn.softplus(v)
    return jax.ops.segment_sum(msg, dst, num_segments=N)


def _prj(Wf, bf, Ws, bs):
    A = jnp.concatenate([Wf[:C], Ws[:C]], axis=1)
    B = jnp.concatenate([Wf[C:2 * C], Ws[C:2 * C]], axis=1)
    Ab = jnp.concatenate([bf, bs]).reshape(1, 2 * C)
    we = jnp.concatenate([Wf[2 * C], Ws[2 * C]]).reshape(1, 2 * C)
    return A, Ab, B, we


def kernel(x, edge_index, edge_attr, batch, W0, b0, g0, be0, Wf1, bf1, Ws1,
           bs1, g1, be1, Wf2, bf2, Ws2, bs2, g2, be2, Wf3, bf3, Ws3, bs3, g3,
           be3, fcW1, fcb1, fcW2, fcb2):
    src = edge_index[0]
    dst = edge_index[1]
    e = edge_attr[:, 0]

    layers = [(Wf1, bf1, Ws1, bs1), (Wf2, bf2, Ws2, bs2), (Wf3, bf3, Ws3, bs3)]
    bn = [(g0, be0), (g1, be1), (g2, be2), (g3, be3)]

    y0 = _lin0(x, W0, b0)
    mu, rstd = _moments(y0)
    A, Ab, B, we = _prj(*layers[0])
    h, P, Q = _update(y0, None, mu, rstd, bn[0][0].reshape(1, C),
                      bn[0][1].reshape(1, C), A, Ab, B, True)

    for i in (1, 2, 3):
        agg = _edge_pass(P, Q, we, src, dst, e)
        mu, rstd = _moments(agg)
        g, be = bn[i]
        if i < 3:
            A, Ab, B, we = _prj(*layers[i])
            h, P, Q = _update(agg, h, mu, rstd, g.reshape(1, C),
                              be.reshape(1, C), A, Ab, B, True)
        else:
            return _pool_head(agg, h, mu, rstd, g.reshape(1, C),
                              be.reshape(1, C), batch, fcW1, fcb1, fcW2, fcb2)


# speedup vs baseline: 1.3609x; 1.3546x over previous
import functools
import jax
import jax.numpy as jnp
from jax import lax
from jax.experimental import pallas as pl
from jax.experimental.pallas import tpu as pltpu
from jax.experimental.pallas import tpu_sc as plsc

N = 100000
E = 1600000
G = 256
C = 64
RB = 2000
GRID = N // RB

NC = 2
NS = 16
NW = NC * NS
KR = 4
N_PAD = 100352
R = N_PAD // KR
RT = R // NS
E_PAD = 1601536
EW = E_PAD // NW
CHUNK = 128
NCH_W = EW // CHUNK
NCH_T = E_PAD // CHUNK // NS
SENT = 2 ** 30
DPAD = 100351

LOG1P_C = (0.030102247599621303, -0.1301179302884263, 0.2833023836204729,
           -0.4891557820115131, 0.9990102089269793, 2.2132783999840164e-05)

BW = 512
NBUCK = 196
NBJ = NBUCK + 1
AGG_ROWS = NBUCK * BW
EB = 2048
EGRID = E_PAD // EB
E_PAD3 = E_PAD + 256
RND = 7
NCH_MAX = E_PAD // CHUNK


def _stats_body(y_ref, mu_ref, out_ref):
    i = pl.program_id(0)

    @pl.when(i == 0)
    def _():
        out_ref[...] = jnp.zeros_like(out_ref)

    y = y_ref[...]
    yc = y - mu_ref[...]
    s1 = jnp.sum(y, axis=0, keepdims=True)
    s2 = jnp.sum(yc * yc, axis=0, keepdims=True)
    out_ref[...] += jnp.concatenate([s1, s2], axis=0)


def _stats(y, mu):
    return pl.pallas_call(
        _stats_body,
        grid=(GRID,),
        in_specs=[pl.BlockSpec((RB, C), lambda i: (i, 0)),
                  pl.BlockSpec((1, C), lambda i: (0, 0))],
        out_specs=pl.BlockSpec((2, C), lambda i: (0, 0)),
        out_shape=jax.ShapeDtypeStruct((2, C), jnp.float32),
    )(y, mu)


def _moments(y):
    s = _stats(y, jnp.zeros((1, C), jnp.float32))
    mu = (s[0] / N).reshape(1, C)
    s2 = _stats(y, mu)
    var = s2[1] / N
    rstd = 1.0 / jnp.sqrt(var + 1e-5)
    return mu, rstd.reshape(1, C)


def _lin0_body(x_ref, w_ref, b_ref, y_ref):
    y_ref[...] = jnp.dot(x_ref[...], w_ref[...],
                         preferred_element_type=jnp.float32) + b_ref[...]


def _lin0(x, W0, b0):
    return pl.pallas_call(
        _lin0_body,
        grid=(GRID,),
        in_specs=[
            pl.BlockSpec((RB, 12), lambda i: (i, 0)),
            pl.BlockSpec((12, C), lambda i: (0, 0)),
            pl.BlockSpec((1, C), lambda i: (0, 0)),
        ],
        out_specs=pl.BlockSpec((RB, C), lambda i: (i, 0)),
        out_shape=jax.ShapeDtypeStruct((N, C), jnp.float32),
    )(x, W0, b0.reshape(1, C))


def _update_body(use_relu, has_res, y_ref, res_ref, mu_ref, rstd_ref, g_ref,
                 be_ref, A_ref, Ab_ref, B_ref, h_ref, p_ref, q_ref):
    h = (y_ref[...] - mu_ref[...]) * (g_ref[...] * rstd_ref[...]) + be_ref[...]
    if has_res:
        h = h + res_ref[...]
    if use_relu:
        h = jnp.maximum(h, 0.0)
    h_ref[...] = h
    p_ref[...] = jnp.dot(h, A_ref[...],
                         preferred_element_type=jnp.float32) + Ab_ref[...]
    q_ref[...] = jnp.dot(h, B_ref[...], preferred_element_type=jnp.float32)


def _update(y, res, mu, rstd, g, be, A, Ab, B, use_relu):
    has_res = res is not None
    if not has_res:
        res = jnp.zeros((1, C), jnp.float32)
    body = functools.partial(_update_body, use_relu, has_res)
    res_spec = (pl.BlockSpec((RB, C), lambda i: (i, 0)) if has_res
                else pl.BlockSpec((1, C), lambda i: (0, 0)))
    return pl.pallas_call(
        body,
        grid=(GRID,),
        in_specs=[
            pl.BlockSpec((RB, C), lambda i: (i, 0)),
            res_spec,
            pl.BlockSpec((1, C), lambda i: (0, 0)),
            pl.BlockSpec((1, C), lambda i: (0, 0)),
            pl.BlockSpec((1, C), lambda i: (0, 0)),
            pl.BlockSpec((1, C), lambda i: (0, 0)),
            pl.BlockSpec((C, 2 * C), lambda i: (0, 0)),
            pl.BlockSpec((1, 2 * C), lambda i: (0, 0)),
            pl.BlockSpec((C, 2 * C), lambda i: (0, 0)),
        ],
        out_specs=[
            pl.BlockSpec((RB, C), lambda i: (i, 0)),
            pl.BlockSpec((RB, 2 * C), lambda i: (i, 0)),
            pl.BlockSpec((RB, 2 * C), lambda i: (i, 0)),
        ],
        out_shape=[
            jax.ShapeDtypeStruct((N, C), jnp.float32),
            jax.ShapeDtypeStruct((N, 2 * C), jnp.float32),
            jax.ShapeDtypeStruct((N, 2 * C), jnp.float32),
        ],
    )(y, res, mu, rstd, g, be, A, Ab, B)


def _pool_body(agg_ref, h2_ref, mu_ref, rstd_ref, g_ref, be_ref, batch_ref,
               fcW1_ref, fcb1_ref, fcW2_ref, fcb2_ref, out_ref,
               acc_ref, cnt_ref):
    i = pl.program_id(0)

    @pl.when(i == 0)
    def _():
        acc_ref[...] = jnp.zeros_like(acc_ref)
        cnt_ref[...] = jnp.zeros_like(cnt_ref)

    h3 = (agg_ref[...] - mu_ref[...]) * (g_ref[...] * rstd_ref[...]) \
        + be_ref[...] + h2_ref[...]
    oh = (batch_ref[...] ==
          jax.lax.broadcasted_iota(jnp.int32, (1, G), 1)).astype(jnp.float32)
    acc_ref[...] += jax.lax.dot_general(
        oh, h3, (((0,), (0,)), ((), ())), preferred_element_type=jnp.float32)
    cnt_ref[...] += jnp.sum(oh, axis=0, keepdims=True)

    @pl.when(i == GRID - 1)
    def _():
        pooled = acc_ref[...] / jnp.maximum(cnt_ref[...], 1.0).T
        t = jnp.maximum(
            jnp.dot(pooled, fcW1_ref[...], preferred_element_type=jnp.float32)
            + fcb1_ref[...], 0.0)
        out_ref[...] = jnp.dot(t, fcW2_ref[...],
                               preferred_element_type=jnp.float32) + fcb2_ref[...]


def _pool_head(agg3, h2, mu, rstd, g, be, batch, fcW1, fcb1, fcW2, fcb2):
    return pl.pallas_call(
        _pool_body,
        grid=(GRID,),
        in_specs=[
            pl.BlockSpec((RB, C), lambda i: (i, 0)),
            pl.BlockSpec((RB, C), lambda i: (i, 0)),
            pl.BlockSpec((1, C), lambda i: (0, 0)),
            pl.BlockSpec((1, C), lambda i: (0, 0)),
            pl.BlockSpec((1, C), lambda i: (0, 0)),
            pl.BlockSpec((1, C), lambda i: (0, 0)),
            pl.BlockSpec((RB, 1), lambda i: (i, 0)),
            pl.BlockSpec((C, 32), lambda i: (0, 0)),
            pl.BlockSpec((1, 32), lambda i: (0, 0)),
            pl.BlockSpec((32, 1), lambda i: (0, 0)),
            pl.BlockSpec((1, 1), lambda i: (0, 0)),
        ],
        out_specs=pl.BlockSpec((G, 1), lambda i: (0, 0)),
        out_shape=jax.ShapeDtypeStruct((G, 1), jnp.float32),
        scratch_shapes=[pltpu.VMEM((G, C), jnp.float32),
                        pltpu.VMEM((1, G), jnp.float32)],
    )(agg3, h2, mu, rstd, g, be, batch.reshape(N, 1), fcW1,
      fcb1.reshape(1, 32), fcW2, fcb2.reshape(1, 1))


def _bucket_of(d):
    return jnp.minimum(jax.lax.shift_right_logical(d, 9), NBUCK - 1)


def _bcount_body(d_ref, out_ref):
    b = _bucket_of(d_ref[...])
    oh = (b == jax.lax.broadcasted_iota(jnp.int32, (1, NBUCK), 1))
    out_ref[...] = jnp.sum(oh.astype(jnp.float32), axis=0,
                           keepdims=True).reshape(1, 1, NBUCK)


def _bcount(dsts2):
    return pl.pallas_call(
        _bcount_body,
        grid=(EGRID,),
        in_specs=[pl.BlockSpec((EB, 1), lambda i: (i, 0))],
        out_specs=pl.BlockSpec((1, 1, NBUCK), lambda i: (i, 0, 0)),
        out_shape=jax.ShapeDtypeStruct((EGRID, 1, NBUCK), jnp.float32),
    )(dsts2)


def _slots_body(d_ref, off_ref, slot_ref):
    b = _bucket_of(d_ref[...])
    oh = (b == jax.lax.broadcasted_iota(jnp.int32, (1, NBUCK), 1)
          ).astype(jnp.float32)
    oh3 = oh.reshape(EB // 8, 8, NBUCK)
    run = jnp.zeros((EB // 8, NBUCK), jnp.float32)
    parts = []
    for r in range(8):
        parts.append(run[:, None, :])
        run = run + oh3[:, r, :]
    excl8 = jnp.concatenate(parts, axis=1)
    inc = run
    sh = 1
    while sh < EB // 8:
        inc = inc + jnp.concatenate(
            [jnp.zeros((sh, NBUCK), jnp.float32), inc[:-sh]], axis=0)
        sh *= 2
    rank3 = excl8 + (inc - run)[:, None, :]
    rank = jnp.sum((rank3.reshape(EB, NBUCK)
                    + off_ref[...].reshape(1, NBUCK)) * oh, axis=1)
    slot_ref[...] = rank.astype(jnp.int32)[:, None]


def _slots(dsts2, off):
    return pl.pallas_call(
        _slots_body,
        grid=(EGRID,),
        in_specs=[pl.BlockSpec((EB, 1), lambda i: (i, 0)),
                  pl.BlockSpec((1, 1, NBUCK), lambda i: (i, 0, 0))],
        out_specs=pl.BlockSpec((EB, 1), lambda i: (i, 0)),
        out_shape=jax.ShapeDtypeStruct((E_PAD, 1), jnp.int32),
    )(dsts2, off)


def _sc_permute(dstsp, srcp, ep, slots):
    mesh = plsc.VectorSubcoreMesh(core_axis_name="c", subcore_axis_name="s")

    @functools.partial(
        pl.kernel, mesh=mesh,
        out_type=[jax.ShapeDtypeStruct((E_PAD3,), jnp.int32),
                  jax.ShapeDtypeStruct((E_PAD3,), jnp.int32),
                  jax.ShapeDtypeStruct((E_PAD3,), jnp.float32)],
        scratch_types=[
            pltpu.VMEM((1, CHUNK), jnp.int32),
            pltpu.VMEM((CHUNK,), jnp.int32),
            pltpu.VMEM((CHUNK,), jnp.int32),
            pltpu.VMEM((CHUNK,), jnp.float32),
            pltpu.VMEM((256,), jnp.int32),
            pltpu.VMEM((256,), jnp.float32),
            pltpu.SemaphoreType.DMA,
            pltpu.SemaphoreType.DMA,
            pltpu.SemaphoreType.DMA,
        ])
    def k(dsts_h, srcp_h, ep_h, slots_h, db_h, sb_h, eb_h,
          sltv, dv, sv, evb, tl_i, tl_f, sem1, sem2, sem3):
        wid = lax.axis_index("s") * NC + lax.axis_index("c")

        def chunk_body(ch, carry):
            off = wid * EW + ch * CHUNK
            slt = sltv.at[0]
            pltpu.sync_copy(slots_h.at[pl.ds(off, CHUNK)], slt)
            pltpu.sync_copy(dsts_h.at[pl.ds(off, CHUNK)], dv)
            pltpu.sync_copy(srcp_h.at[pl.ds(off, CHUNK)], sv)
            pltpu.sync_copy(ep_h.at[pl.ds(off, CHUNK)], evb)
            c1 = pltpu.async_copy(dv, db_h.at[slt], sem1)
            c2 = pltpu.async_copy(sv, sb_h.at[slt], sem2)
            c3 = pltpu.async_copy(evb, eb_h.at[slt], sem3)
            c1.wait()
            c2.wait()
            c3.wait()
            return carry

        lax.fori_loop(0, NCH_W, chunk_body, 0)

        @pl.when(wid == 0)
        def _():
            for i in range(16):
                tl_i[pl.ds(i * 16, 16)] = jnp.full((16,), DPAD, jnp.int32)
                tl_f[pl.ds(i * 16, 16)] = jnp.zeros((16,), jnp.float32)
            pltpu.sync_copy(tl_i, db_h.at[pl.ds(E_PAD, 256)])
            pltpu.sync_copy(tl_i, sb_h.at[pl.ds(E_PAD, 256)])
            pltpu.sync_copy(tl_f, eb_h.at[pl.ds(E_PAD, 256)])

    return k(dstsp, srcp, ep, slots)


def _sc_layer(Pp, Qp, dstb, srcb, eb, bounds, zrow, wev):
    mesh = plsc.VectorSubcoreMesh(core_axis_name="c", subcore_axis_name="s")

    @functools.partial(
        pl.kernel, mesh=mesh,
        out_type=jax.ShapeDtypeStruct((AGG_ROWS, C), jnp.float32),
        scratch_types=[
            pltpu.VMEM((RND * 16,), jnp.int32),
            pltpu.VMEM((CHUNK,), jnp.int32),
            pltpu.VMEM((CHUNK,), jnp.int32),
            pltpu.VMEM((CHUNK,), jnp.int32),
            pltpu.VMEM((CHUNK,), jnp.float32),
            pltpu.VMEM((CHUNK, 2 * C), jnp.float32),
            pltpu.VMEM((CHUNK, 2 * C), jnp.float32),
            pltpu.VMEM((CHUNK, C), jnp.float32),
            pltpu.VMEM((BW + 8, C), jnp.float32),
            pltpu.VMEM((8, 16), jnp.float32),
            pltpu.SemaphoreType.DMA,
            pltpu.SemaphoreType.DMA,
        ])
    def k(Pp_h, Qp_h, dstb_h, srcb_h, eb_h, bounds_h, z_h, wev_h, agg_h,
          bnd, dstv, idxg, lidx, evb, Pbuf, Qbuf, msgb, aggt, wevv,
          sem1, sem2):
        tid = lax.axis_index("s") * NC + lax.axis_index("c")
        pltpu.sync_copy(wev_h, wevv)
        pltpu.sync_copy(bounds_h.at[pl.ds(tid * (RND * 16), RND * 16)], bnd)
        w8 = tuple(wevv[i] for i in range(8))

        def round_body(rnd, w8r):
            bv = bnd[pl.ds(rnd * 16, 16)]
            bkt = bv[0]
            s = bv[1]
            e_ = bv[2]
            bbase = bkt * BW
            c0 = pl.multiple_of(jnp.bitwise_and(s, jnp.int32(-8)), 8)
            nch = jax.lax.shift_right_logical(e_ - c0 + 127, 7)

            @pl.when(bkt < NBUCK)
            def _():
                pltpu.sync_copy(z_h, aggt)

            def chunk_body(ch, w8c):
                @pl.when(ch < nch)
                def _():
                    off = c0 + ch * CHUNK
                    pltpu.sync_copy(dstb_h.at[pl.ds(off, CHUNK)], dstv)
                    pltpu.sync_copy(srcb_h.at[pl.ds(off, CHUNK)], idxg)
                    pltpu.sync_copy(eb_h.at[pl.ds(off, CHUNK)], evb)
                    for g in range(8):
                        d = dstv[pl.ds(g * 16, 16)]
                        inb = (d >= bbase) & (d < bbase + BW)
                        lidx[pl.ds(g * 16, 16)] = jnp.where(inb, d - bbase, BW)
                    cp1 = pltpu.async_copy(Pp_h.at[dstv], Pbuf, sem1)
                    cp2 = pltpu.async_copy(Qp_h.at[idxg], Qbuf, sem2)
                    cp1.wait()
                    cp2.wait()

                    @plsc.parallel_loop(0, CHUNK // 16, carry=w8c)
                    def edge_body(grp, w8i):
                        ev16 = evb[pl.ds(grp * 16, 16)]
                        for ll in range(16):
                            j = grp * 16 + ll
                            esp = lax.broadcast_in_dim(ev16[ll], (16,), ())
                            for cg in range(4):
                                u = (Pbuf[j, pl.ds(cg * 16, 16)]
                                     + Qbuf[j, pl.ds(cg * 16, 16)]
                                     + esp * w8i[cg])
                                sig = 1.0 / (1.0 + jnp.exp(-u))
                                v = (Pbuf[j, pl.ds(C + cg * 16, 16)]
                                     + Qbuf[j, pl.ds(C + cg * 16, 16)]
                                     + esp * w8i[4 + cg])
                                t = jnp.exp(-jnp.abs(v))
                                p = LOG1P_C[0]
                                for cc in LOG1P_C[1:]:
                                    p = p * t + cc
                                sp = jnp.maximum(v, 0.0) + p
                                msgb[j, pl.ds(cg * 16, 16)] = sig * sp
                        return w8i

                    def acc_body(grp, cacc):
                        l16 = lidx[pl.ds(grp * 16, 16)]
                        for ll in range(16):
                            j = grp * 16 + ll
                            lr = l16[ll]
                            for cg in range(4):
                                plsc.addupdate(
                                    aggt.at[lr, pl.ds(cg * 16, 16)],
                                    msgb[j, pl.ds(cg * 16, 16)])
                        return cacc

                    lax.fori_loop(0, CHUNK // 16, acc_body, 0)

                return w8c

            lax.fori_loop(0, NCH_MAX, chunk_body, w8r)

            @pl.when(bkt < NBUCK)
            def _():
                pltpu.sync_copy(aggt.at[pl.ds(0, BW)],
                                agg_h.at[pl.ds(bbase, BW)])

            return w8r

        lax.fori_loop(0, RND, round_body, w8)

    return k(Pp, Qp, dstb, srcb, eb, bounds, zrow, wev)


def _sc_msg(Pp, Qp, srcp, dstgp, ep, wev):
    mesh = plsc.VectorSubcoreMesh(core_axis_name="c", subcore_axis_name="s")

    @functools.partial(
        pl.kernel, mesh=mesh,
        out_type=jax.ShapeDtypeStruct((E_PAD, C), jnp.float32),
        scratch_types=[
            pltpu.VMEM((CHUNK,), jnp.int32),
            pltpu.VMEM((CHUNK,), jnp.int32),
            pltpu.VMEM((CHUNK,), jnp.float32),
            pltpu.VMEM((CHUNK, 2 * C), jnp.float32),
            pltpu.VMEM((CHUNK, 2 * C), jnp.float32),
            pltpu.VMEM((CHUNK, C), jnp.float32),
            pltpu.VMEM((8, 16), jnp.float32),
            pltpu.SemaphoreType.DMA,
            pltpu.SemaphoreType.DMA,
        ])
    def k(Pp_h, Qp_h, srcp_h, dstgp_h, ep_h, wev_h, msg_h,
          idxs, idxd, evb, Pbuf, Qbuf, msgb, wevv, sem1, sem2):
        wid = lax.axis_index("s") * NC + lax.axis_index("c")
        pltpu.sync_copy(wev_h, wevv)
        w8 = tuple(wevv[i] for i in range(8))

        def chunk_body(ch, w8c):
            off = wid * EW + ch * CHUNK
            pltpu.sync_copy(srcp_h.at[pl.ds(off, CHUNK)], idxs)
            pltpu.sync_copy(dstgp_h.at[pl.ds(off, CHUNK)], idxd)
            pltpu.sync_copy(ep_h.at[pl.ds(off, CHUNK)], evb)
            cp1 = pltpu.async_copy(Pp_h.at[idxd], Pbuf, sem1)
            cp2 = pltpu.async_copy(Qp_h.at[idxs], Qbuf, sem2)
            cp1.wait()
            cp2.wait()

            @plsc.parallel_loop(0, CHUNK // 16, carry=w8c)
            def edge_body(grp, w8i):
                ev16 = evb[pl.ds(grp * 16, 16)]
                for ll in range(16):
                    j = grp * 16 + ll
                    esp = lax.broadcast_in_dim(ev16[ll], (16,), ())
                    for cg in range(4):
                        u = (Pbuf[j, pl.ds(cg * 16, 16)]
                             + Qbuf[j, pl.ds(cg * 16, 16)] + esp * w8i[cg])
                        sig = 1.0 / (1.0 + jnp.exp(-u))
                        v = (Pbuf[j, pl.ds(C + cg * 16, 16)]
                             + Qbuf[j, pl.ds(C + cg * 16, 16)]
                             + esp * w8i[4 + cg])
                        t = jnp.exp(-jnp.abs(v))
                        p = LOG1P_C[0]
                        for cc in LOG1P_C[1:]:
                            p = p * t + cc
                        sp = jnp.maximum(v, 0.0) + p
                        msgb[j, pl.ds(cg * 16, 16)] = sig * sp
                return w8i

            pltpu.sync_copy(msgb, msg_h.at[pl.ds(off, CHUNK)])
            return w8c

        lax.fori_loop(0, NCH_W, chunk_body, w8)

    return k(Pp, Qp, srcp, dstgp, ep, wev)


def _sc_scatter(msg, dstsp, zrows):
    mesh = plsc.VectorSubcoreMesh(core_axis_name="c", subcore_axis_name="s")

    @functools.partial(
        pl.kernel, mesh=mesh,
        out_type=jax.ShapeDtypeStruct((N_PAD, C), jnp.float32),
        scratch_types=[
            pltpu.VMEM((CHUNK,), jnp.int32),
            pltpu.VMEM((CHUNK,), jnp.int32),
            pltpu.VMEM((CHUNK, C), jnp.float32),
            pltpu.VMEM_SHARED((R + 8, C), jnp.float32),
        ])
    def k(msg_h, dsts_h, z_h, agg_h, dstv, lidx, msgb, aggsh):
        cid = lax.axis_index("c")
        sid = lax.axis_index("s")
        for rnd in range(KR // NC):
            base = (cid * (KR // NC) + rnd) * R
            pltpu.sync_copy(z_h.at[pl.ds(0, RT)],
                            aggsh.at[pl.ds(sid * RT, RT)])

            @pl.when(sid == 0)
            def _():
                pltpu.sync_copy(z_h.at[pl.ds(0, 8)], aggsh.at[pl.ds(R, 8)])

            plsc.subcore_barrier()

            def chunk_body(ch, carry):
                off = (sid * NCH_T + ch) * CHUNK
                pltpu.sync_copy(dsts_h.at[pl.ds(off, CHUNK)], dstv)
                pltpu.sync_copy(msg_h.at[pl.ds(off, CHUNK)], msgb)
                for g in range(8):
                    d = dstv[pl.ds(g * 16, 16)]
                    inr = (d >= base) & (d < base + R)
                    lidx[pl.ds(g * 16, 16)] = jnp.where(inr, d - base, R)
                pltpu.sync_copy(msgb, aggsh.at[lidx], add=True)
                return carry

            lax.fori_loop(0, NCH_T, chunk_body, 0)
            plsc.subcore_barrier()
            pltpu.sync_copy(aggsh.at[pl.ds(sid * RT, RT)],
                            agg_h.at[pl.ds(base + sid * RT, RT)])
            plsc.subcore_barrier()

    return k(msg, dstsp, zrows)


def _bucketize(srcp, dstsp, ep):
    dsts2 = dstsp.reshape(E_PAD, 1)
    counts = _bcount(dsts2).reshape(EGRID, NBUCK)
    blk_excl = jnp.cumsum(counts, axis=0) - counts
    tot = jnp.sum(counts, axis=0)
    bstart = jnp.cumsum(tot) - tot
    off = bstart[None, :] + blk_excl
    slots = _slots(dsts2, off.reshape(EGRID, 1, NBUCK))
    dstb, srcb, ebkt = _sc_permute(dstsp, srcp, ep, slots.reshape(E_PAD))

    bs = bstart.astype(jnp.int32)
    be = (bstart + tot).astype(jnp.int32)
    t = jnp.arange(NW, dtype=jnp.int32)
    rows = []
    for r in range(RND):
        if r < 6:
            b = r * NW + t
        else:
            b = jnp.where(t < NBUCK - 6 * NW, 6 * NW + t, NBUCK)
        bc = jnp.minimum(b, NBUCK - 1)
        sr = jnp.where(b < NBUCK, bs[bc], 0)
        er = jnp.where(b < NBUCK, be[bc], 0)
        rows.append(jnp.stack([b, sr, er], axis=-1))
    bnd3 = jnp.stack(rows, axis=1).reshape(NW * RND, 3)
    bounds = jnp.pad(bnd3, ((0, 0), (0, 13))).reshape(-1)
    return dstb, srcb, ebkt, bounds


_BISECT_JNP_LAYER = False


def _edge_pass(Pp, Qp, we, dstb, srcb, ebkt, bounds, zrow):
    Pp = jnp.pad(Pp, ((0, N_PAD - N), (0, 0)))
    Qp = jnp.pad(Qp, ((0, N_PAD - N), (0, 0)))
    if _BISECT_JNP_LAYER:
        db = dstb[:E_PAD]
        valid = (db < N)[:, None].astype(jnp.float32)
        dc = jnp.minimum(db, N - 1)
        Z = Pp[dc] + Qp[srcb[:E_PAD]] + ebkt[:E_PAD][:, None] * we
        msg = jax.nn.sigmoid(Z[:, :C]) * jax.nn.softplus(Z[:, C:]) * valid
        return jax.ops.segment_sum(msg, dc, num_segments=N)
    agg = _sc_layer(Pp, Qp, dstb, srcb, ebkt, bounds, zrow,
                    we.reshape(8, 16))
    return agg[:N]


def _prj(Wf, bf, Ws, bs):
    A = jnp.concatenate([Wf[:C], Ws[:C]], axis=1)
    B = jnp.concatenate([Wf[C:2 * C], Ws[C:2 * C]], axis=1)
    Ab = jnp.concatenate([bf, bs]).reshape(1, 2 * C)
    we = jnp.concatenate([Wf[2 * C], Ws[2 * C]]).reshape(1, 2 * C)
    return A, Ab, B, we


def kernel(x, edge_index, edge_attr, batch, W0, b0, g0, be0, Wf1, bf1, Ws1,
           bs1, g1, be1, Wf2, bf2, Ws2, bs2, g2, be2, Wf3, bf3, Ws3, bs3, g3,
           be3, fcW1, fcb1, fcW2, fcb2):
    src = edge_index[0]
    dst = edge_index[1]
    e = edge_attr[:, 0]

    srcp = jnp.pad(src, (0, E_PAD - E))
    dstsp = jnp.pad(dst, (0, E_PAD - E), constant_values=DPAD)
    ep = jnp.pad(e, (0, E_PAD - E))
    dstb, srcb, ebkt, bounds = _bucketize(srcp, dstsp, ep)
    zrow = jnp.zeros((BW + 8, C), jnp.float32)

    layers = [(Wf1, bf1, Ws1, bs1), (Wf2, bf2, Ws2, bs2), (Wf3, bf3, Ws3, bs3)]
    bn = [(g0, be0), (g1, be1), (g2, be2), (g3, be3)]

    y0 = _lin0(x, W0, b0)
    mu, rstd = _moments(y0)
    A, Ab, B, we = _prj(*layers[0])
    h, P, Q = _update(y0, None, mu, rstd, bn[0][0].reshape(1, C),
                      bn[0][1].reshape(1, C), A, Ab, B, True)

    for i in (1, 2, 3):
        agg = _edge_pass(P, Q, we, dstb, srcb, ebkt, bounds, zrow)
        mu, rstd = _moments(agg)
        g, be = bn[i]
        if i < 3:
            A, Ab, B, we = _prj(*layers[i])
            h, P, Q = _update(agg, h, mu, rstd, g.reshape(1, C),
                              be.reshape(1, C), A, Ab, B, True)
        else:
            return _pool_head(agg, h, mu, rstd, g.reshape(1, C),
                              be.reshape(1, C), batch, fcW1, fcb1, fcW2, fcb2)


# speedup vs baseline: 1.3839x; 1.0169x over previous
import functools
import jax
import jax.numpy as jnp
from jax import lax
from jax.experimental import pallas as pl
from jax.experimental.pallas import tpu as pltpu
from jax.experimental.pallas import tpu_sc as plsc

N = 100000
E = 1600000
G = 256
C = 64
RB = 2000
GRID = N // RB

NC = 2
NS = 16
NW = NC * NS
KR = 4
N_PAD = 100352
R = N_PAD // KR
RT = R // NS
E_PAD = 1601536
EW = E_PAD // NW
CHUNK = 128
NCH_W = EW // CHUNK
NCH_T = E_PAD // CHUNK // NS
SENT = 2 ** 30
DPAD = 100351

LOG1P_C = (0.030102247599621303, -0.1301179302884263, 0.2833023836204729,
           -0.4891557820115131, 0.9990102089269793, 2.2132783999840164e-05)

BW = 512
NBUCK = 196
NBJ = NBUCK + 1
AGG_ROWS = NBUCK * BW
EB = 2048
EGRID = E_PAD // EB
E_PAD3 = E_PAD + 256
RND = 7
NCH_MAX = E_PAD // CHUNK


def _stats_body(y_ref, mu_ref, out_ref):
    i = pl.program_id(0)

    @pl.when(i == 0)
    def _():
        out_ref[...] = jnp.zeros_like(out_ref)

    y = y_ref[...]
    yc = y - mu_ref[...]
    s1 = jnp.sum(y, axis=0, keepdims=True)
    s2 = jnp.sum(yc * yc, axis=0, keepdims=True)
    out_ref[...] += jnp.concatenate([s1, s2], axis=0)


def _stats(y, mu):
    return pl.pallas_call(
        _stats_body,
        grid=(GRID,),
        in_specs=[pl.BlockSpec((RB, C), lambda i: (i, 0)),
                  pl.BlockSpec((1, C), lambda i: (0, 0))],
        out_specs=pl.BlockSpec((2, C), lambda i: (0, 0)),
        out_shape=jax.ShapeDtypeStruct((2, C), jnp.float32),
    )(y, mu)


def _moments(y):
    s = _stats(y, jnp.zeros((1, C), jnp.float32))
    mu = (s[0] / N).reshape(1, C)
    s2 = _stats(y, mu)
    var = s2[1] / N
    rstd = 1.0 / jnp.sqrt(var + 1e-5)
    return mu, rstd.reshape(1, C)


def _lin0_body(x_ref, w_ref, b_ref, y_ref):
    y_ref[...] = jnp.dot(x_ref[...], w_ref[...],
                         preferred_element_type=jnp.float32) + b_ref[...]


def _lin0(x, W0, b0):
    return pl.pallas_call(
        _lin0_body,
        grid=(GRID,),
        in_specs=[
            pl.BlockSpec((RB, 12), lambda i: (i, 0)),
            pl.BlockSpec((12, C), lambda i: (0, 0)),
            pl.BlockSpec((1, C), lambda i: (0, 0)),
        ],
        out_specs=pl.BlockSpec((RB, C), lambda i: (i, 0)),
        out_shape=jax.ShapeDtypeStruct((N, C), jnp.float32),
    )(x, W0, b0.reshape(1, C))


def _update_body(use_relu, has_res, y_ref, res_ref, mu_ref, rstd_ref, g_ref,
                 be_ref, A_ref, Ab_ref, B_ref, h_ref, p_ref, q_ref):
    h = (y_ref[...] - mu_ref[...]) * (g_ref[...] * rstd_ref[...]) + be_ref[...]
    if has_res:
        h = h + res_ref[...]
    if use_relu:
        h = jnp.maximum(h, 0.0)
    h_ref[...] = h
    p_ref[...] = jnp.dot(h, A_ref[...],
                         preferred_element_type=jnp.float32) + Ab_ref[...]
    q_ref[...] = jnp.dot(h, B_ref[...], preferred_element_type=jnp.float32)


def _update(y, res, mu, rstd, g, be, A, Ab, B, use_relu):
    has_res = res is not None
    if not has_res:
        res = jnp.zeros((1, C), jnp.float32)
    body = functools.partial(_update_body, use_relu, has_res)
    res_spec = (pl.BlockSpec((RB, C), lambda i: (i, 0)) if has_res
                else pl.BlockSpec((1, C), lambda i: (0, 0)))
    return pl.pallas_call(
        body,
        grid=(GRID,),
        in_specs=[
            pl.BlockSpec((RB, C), lambda i: (i, 0)),
            res_spec,
            pl.BlockSpec((1, C), lambda i: (0, 0)),
            pl.BlockSpec((1, C), lambda i: (0, 0)),
            pl.BlockSpec((1, C), lambda i: (0, 0)),
            pl.BlockSpec((1, C), lambda i: (0, 0)),
            pl.BlockSpec((C, 2 * C), lambda i: (0, 0)),
            pl.BlockSpec((1, 2 * C), lambda i: (0, 0)),
            pl.BlockSpec((C, 2 * C), lambda i: (0, 0)),
        ],
        out_specs=[
            pl.BlockSpec((RB, C), lambda i: (i, 0)),
            pl.BlockSpec((RB, 2 * C), lambda i: (i, 0)),
            pl.BlockSpec((RB, 2 * C), lambda i: (i, 0)),
        ],
        out_shape=[
            jax.ShapeDtypeStruct((N, C), jnp.float32),
            jax.ShapeDtypeStruct((N, 2 * C), jnp.float32),
            jax.ShapeDtypeStruct((N, 2 * C), jnp.float32),
        ],
    )(y, res, mu, rstd, g, be, A, Ab, B)


def _pool_body(agg_ref, h2_ref, mu_ref, rstd_ref, g_ref, be_ref, batch_ref,
               fcW1_ref, fcb1_ref, fcW2_ref, fcb2_ref, out_ref,
               acc_ref, cnt_ref):
    i = pl.program_id(0)

    @pl.when(i == 0)
    def _():
        acc_ref[...] = jnp.zeros_like(acc_ref)
        cnt_ref[...] = jnp.zeros_like(cnt_ref)

    h3 = (agg_ref[...] - mu_ref[...]) * (g_ref[...] * rstd_ref[...]) \
        + be_ref[...] + h2_ref[...]
    oh = (batch_ref[...] ==
          jax.lax.broadcasted_iota(jnp.int32, (1, G), 1)).astype(jnp.float32)
    acc_ref[...] += jax.lax.dot_general(
        oh, h3, (((0,), (0,)), ((), ())), preferred_element_type=jnp.float32)
    cnt_ref[...] += jnp.sum(oh, axis=0, keepdims=True)

    @pl.when(i == GRID - 1)
    def _():
        pooled = acc_ref[...] / jnp.maximum(cnt_ref[...], 1.0).T
        t = jnp.maximum(
            jnp.dot(pooled, fcW1_ref[...], preferred_element_type=jnp.float32)
            + fcb1_ref[...], 0.0)
        out_ref[...] = jnp.dot(t, fcW2_ref[...],
                               preferred_element_type=jnp.float32) + fcb2_ref[...]


def _pool_head(agg3, h2, mu, rstd, g, be, batch, fcW1, fcb1, fcW2, fcb2):
    return pl.pallas_call(
        _pool_body,
        grid=(GRID,),
        in_specs=[
            pl.BlockSpec((RB, C), lambda i: (i, 0)),
            pl.BlockSpec((RB, C), lambda i: (i, 0)),
            pl.BlockSpec((1, C), lambda i: (0, 0)),
            pl.BlockSpec((1, C), lambda i: (0, 0)),
            pl.BlockSpec((1, C), lambda i: (0, 0)),
            pl.BlockSpec((1, C), lambda i: (0, 0)),
            pl.BlockSpec((RB, 1), lambda i: (i, 0)),
            pl.BlockSpec((C, 32), lambda i: (0, 0)),
            pl.BlockSpec((1, 32), lambda i: (0, 0)),
            pl.BlockSpec((32, 1), lambda i: (0, 0)),
            pl.BlockSpec((1, 1), lambda i: (0, 0)),
        ],
        out_specs=pl.BlockSpec((G, 1), lambda i: (0, 0)),
        out_shape=jax.ShapeDtypeStruct((G, 1), jnp.float32),
        scratch_shapes=[pltpu.VMEM((G, C), jnp.float32),
                        pltpu.VMEM((1, G), jnp.float32)],
    )(agg3, h2, mu, rstd, g, be, batch.reshape(N, 1), fcW1,
      fcb1.reshape(1, 32), fcW2, fcb2.reshape(1, 1))


def _bucket_of(d):
    return jnp.minimum(jax.lax.shift_right_logical(d, 9), NBUCK - 1)


def _bcount_body(d_ref, out_ref):
    b = _bucket_of(d_ref[...])
    oh = (b == jax.lax.broadcasted_iota(jnp.int32, (1, NBUCK), 1))
    out_ref[...] = jnp.sum(oh.astype(jnp.float32), axis=0,
                           keepdims=True).reshape(1, 1, NBUCK)


def _bcount(dsts2):
    return pl.pallas_call(
        _bcount_body,
        grid=(EGRID,),
        in_specs=[pl.BlockSpec((EB, 1), lambda i: (i, 0))],
        out_specs=pl.BlockSpec((1, 1, NBUCK), lambda i: (i, 0, 0)),
        out_shape=jax.ShapeDtypeStruct((EGRID, 1, NBUCK), jnp.float32),
    )(dsts2)


def _slots_body(d_ref, off_ref, slot_ref):
    b = _bucket_of(d_ref[...])
    oh = (b == jax.lax.broadcasted_iota(jnp.int32, (1, NBUCK), 1)
          ).astype(jnp.float32)
    oh3 = oh.reshape(EB // 8, 8, NBUCK)
    run = jnp.zeros((EB // 8, NBUCK), jnp.float32)
    parts = []
    for r in range(8):
        parts.append(run[:, None, :])
        run = run + oh3[:, r, :]
    excl8 = jnp.concatenate(parts, axis=1)
    inc = run
    sh = 1
    while sh < EB // 8:
        inc = inc + jnp.concatenate(
            [jnp.zeros((sh, NBUCK), jnp.float32), inc[:-sh]], axis=0)
        sh *= 2
    rank3 = excl8 + (inc - run)[:, None, :]
    rank = jnp.sum((rank3.reshape(EB, NBUCK)
                    + off_ref[...].reshape(1, NBUCK)) * oh, axis=1)
    slot_ref[...] = rank.astype(jnp.int32)[:, None]


def _slots(dsts2, off):
    return pl.pallas_call(
        _slots_body,
        grid=(EGRID,),
        in_specs=[pl.BlockSpec((EB, 1), lambda i: (i, 0)),
                  pl.BlockSpec((1, 1, NBUCK), lambda i: (i, 0, 0))],
        out_specs=pl.BlockSpec((EB, 1), lambda i: (i, 0)),
        out_shape=jax.ShapeDtypeStruct((E_PAD, 1), jnp.int32),
    )(dsts2, off)


def _sc_permute(dstsp, srcp, ep, s0, s1, s2):
    mesh = plsc.VectorSubcoreMesh(core_axis_name="c", subcore_axis_name="s")

    @functools.partial(
        pl.kernel, mesh=mesh,
        out_type=[jax.ShapeDtypeStruct((2 * E_PAD3,), jnp.int32),
                  jax.ShapeDtypeStruct((E_PAD3,), jnp.float32)],
        scratch_types=[
            pltpu.VMEM((3, CHUNK), jnp.int32),
            pltpu.VMEM((CHUNK,), jnp.int32),
            pltpu.VMEM((CHUNK,), jnp.int32),
            pltpu.VMEM((CHUNK,), jnp.float32),
            pltpu.VMEM((256,), jnp.int32),
            pltpu.VMEM((256,), jnp.float32),
            pltpu.SemaphoreType.DMA,
            pltpu.SemaphoreType.DMA,
            pltpu.SemaphoreType.DMA,
        ])
    def k(dsts_h, srcp_h, ep_h, s0_h, s1_h, s2_h, edc_h, eb_h,
          sltv, dv, sv, evb, tl_i, tl_f, sem1, sem2, sem3):
        wid = lax.axis_index("s") * NC + lax.axis_index("c")

        def chunk_body(ch, carry):
            off = wid * EW + ch * CHUNK
            pltpu.sync_copy(s0_h.at[pl.ds(off, CHUNK)], sltv.at[0])
            pltpu.sync_copy(s1_h.at[pl.ds(off, CHUNK)], sltv.at[1])
            pltpu.sync_copy(s2_h.at[pl.ds(off, CHUNK)], sltv.at[2])
            pltpu.sync_copy(dsts_h.at[pl.ds(off, CHUNK)], dv)
            pltpu.sync_copy(srcp_h.at[pl.ds(off, CHUNK)], sv)
            pltpu.sync_copy(ep_h.at[pl.ds(off, CHUNK)], evb)
            c1 = pltpu.async_copy(dv, edc_h.at[sltv.at[0]], sem1)
            c2 = pltpu.async_copy(sv, edc_h.at[sltv.at[1]], sem2)
            c3 = pltpu.async_copy(evb, eb_h.at[sltv.at[2]], sem3)
            c1.wait()
            c2.wait()
            c3.wait()
            return carry

        lax.fori_loop(0, NCH_W, chunk_body, 0)

        @pl.when(wid == 0)
        def _():
            for i in range(16):
                val = DPAD if i < 8 else 0
                tl_i[pl.ds(i * 16, 16)] = jnp.full((16,), val, jnp.int32)
                tl_f[pl.ds(i * 16, 16)] = jnp.zeros((16,), jnp.float32)
            pltpu.sync_copy(tl_i, edc_h.at[pl.ds(2 * E_PAD, 256)])
            pltpu.sync_copy(tl_i, edc_h.at[pl.ds(2 * E_PAD + 256, 256)])
            pltpu.sync_copy(tl_f, eb_h.at[pl.ds(E_PAD, 256)])

    return k(dstsp, srcp, ep, s0, s1, s2)


def _sc_layer(Pp, Qp, edc, ebkt, bounds, zrow, wev):
    mesh = plsc.VectorSubcoreMesh(core_axis_name="c", subcore_axis_name="s")

    @functools.partial(
        pl.kernel, mesh=mesh,
        out_type=jax.ShapeDtypeStruct((AGG_ROWS, C), jnp.float32),
        scratch_types=[
            pltpu.VMEM((RND * 16,), jnp.int32),
            pltpu.VMEM((2 * CHUNK,), jnp.int32),
            pltpu.VMEM((CHUNK,), jnp.float32),
            pltpu.VMEM((CHUNK,), jnp.int32),
            pltpu.VMEM((CHUNK, 2 * C), jnp.float32),
            pltpu.VMEM((CHUNK, 2 * C), jnp.float32),
            pltpu.VMEM((CHUNK, C), jnp.float32),
            pltpu.VMEM((BW + 8, C), jnp.float32),
            pltpu.VMEM((8, 16), jnp.float32),
            pltpu.SemaphoreType.DMA,
            pltpu.SemaphoreType.DMA,
        ])
    def k(Pp_h, Qp_h, edc_h, eb_h, bounds_h, z_h, wev_h, agg_h,
          bnd, ebuf, evb, lidx, Pbuf, Qbuf, msgb, aggt, wevv,
          sem1, sem2):
        tid = lax.axis_index("s") * NC + lax.axis_index("c")
        pltpu.sync_copy(wev_h, wevv)
        pltpu.sync_copy(bounds_h.at[pl.ds(tid * (RND * 16), RND * 16)], bnd)
        w8 = tuple(wevv[i] for i in range(8))

        def round_body(rnd, w8r):
            bv = bnd[pl.ds(rnd * 16, 16)]
            bkt = bv[0]
            s = bv[1]
            e_ = bv[2]
            bbase = bkt * BW
            c0 = pl.multiple_of(jnp.bitwise_and(s, jnp.int32(-128)), 128)
            nch = jax.lax.shift_right_logical(e_ - c0 + 127, 7)

            @pl.when(bkt < NBUCK)
            def _():
                pltpu.sync_copy(z_h, aggt)

            def chunk_body(ch, w8c):
                @pl.when(ch < nch)
                def _():
                    off = c0 + ch * CHUNK
                    boff = pl.multiple_of(off * 2, 128)
                    pltpu.sync_copy(edc_h.at[pl.ds(boff, 2 * CHUNK)], ebuf)
                    pltpu.sync_copy(eb_h.at[pl.ds(off, CHUNK)], evb)
                    for g in range(8):
                        d = ebuf[pl.ds(g * 16, 16)]
                        inb = (d >= bbase) & (d < bbase + BW)
                        lidx[pl.ds(g * 16, 16)] = jnp.where(inb, d - bbase, BW)
                    cp1 = pltpu.async_copy(Pp_h.at[ebuf.at[pl.ds(0, CHUNK)]],
                                           Pbuf, sem1)
                    cp2 = pltpu.async_copy(
                        Qp_h.at[ebuf.at[pl.ds(CHUNK, CHUNK)]], Qbuf, sem2)
                    cp1.wait()
                    cp2.wait()

                    @plsc.parallel_loop(0, CHUNK // 16, carry=w8c)
                    def edge_body(grp, w8i):
                        ev16 = evb[pl.ds(grp * 16, 16)]
                        for ll in range(16):
                            j = grp * 16 + ll
                            esp = lax.broadcast_in_dim(ev16[ll], (16,), ())
                            for cg in range(4):
                                u = (Pbuf[j, pl.ds(cg * 16, 16)]
                                     + Qbuf[j, pl.ds(cg * 16, 16)]
                                     + esp * w8i[cg])
                                sig = 1.0 / (1.0 + jnp.exp(-u))
                                v = (Pbuf[j, pl.ds(C + cg * 16, 16)]
                                     + Qbuf[j, pl.ds(C + cg * 16, 16)]
                                     + esp * w8i[4 + cg])
                                t = jnp.exp(-jnp.abs(v))
                                p = LOG1P_C[0]
                                for cc in LOG1P_C[1:]:
                                    p = p * t + cc
                                sp = jnp.maximum(v, 0.0) + p
                                msgb[j, pl.ds(cg * 16, 16)] = sig * sp
                        return w8i

                    def acc_body(grp, cacc):
                        l16 = lidx[pl.ds(grp * 16, 16)]
                        for ll in range(16):
                            j = grp * 16 + ll
                            lr = l16[ll]
                            for cg in range(4):
                                plsc.addupdate(
                                    aggt.at[lr, pl.ds(cg * 16, 16)],
                                    msgb[j, pl.ds(cg * 16, 16)])
                        return cacc

                    lax.fori_loop(0, CHUNK // 16, acc_body, 0)

                return w8c

            lax.fori_loop(0, NCH_MAX, chunk_body, w8r)

            @pl.when(bkt < NBUCK)
            def _():
                pltpu.sync_copy(aggt.at[pl.ds(0, BW)],
                                agg_h.at[pl.ds(bbase, BW)])

            return w8r

        lax.fori_loop(0, RND, round_body, w8)

    return k(Pp, Qp, edc, ebkt, bounds, zrow, wev)


def _sc_msg(Pp, Qp, srcp, dstgp, ep, wev):
    mesh = plsc.VectorSubcoreMesh(core_axis_name="c", subcore_axis_name="s")

    @functools.partial(
        pl.kernel, mesh=mesh,
        out_type=jax.ShapeDtypeStruct((E_PAD, C), jnp.float32),
        scratch_types=[
            pltpu.VMEM((CHUNK,), jnp.int32),
            pltpu.VMEM((CHUNK,), jnp.int32),
            pltpu.VMEM((CHUNK,), jnp.float32),
            pltpu.VMEM((CHUNK, 2 * C), jnp.float32),
            pltpu.VMEM((CHUNK, 2 * C), jnp.float32),
            pltpu.VMEM((CHUNK, C), jnp.float32),
            pltpu.VMEM((8, 16), jnp.float32),
            pltpu.SemaphoreType.DMA,
            pltpu.SemaphoreType.DMA,
        ])
    def k(Pp_h, Qp_h, srcp_h, dstgp_h, ep_h, wev_h, msg_h,
          idxs, idxd, evb, Pbuf, Qbuf, msgb, wevv, sem1, sem2):
        wid = lax.axis_index("s") * NC + lax.axis_index("c")
        pltpu.sync_copy(wev_h, wevv)
        w8 = tuple(wevv[i] for i in range(8))

        def chunk_body(ch, w8c):
            off = wid * EW + ch * CHUNK
            pltpu.sync_copy(srcp_h.at[pl.ds(off, CHUNK)], idxs)
            pltpu.sync_copy(dstgp_h.at[pl.ds(off, CHUNK)], idxd)
            pltpu.sync_copy(ep_h.at[pl.ds(off, CHUNK)], evb)
            cp1 = pltpu.async_copy(Pp_h.at[idxd], Pbuf, sem1)
            cp2 = pltpu.async_copy(Qp_h.at[idxs], Qbuf, sem2)
            cp1.wait()
            cp2.wait()

            @plsc.parallel_loop(0, CHUNK // 16, carry=w8c)
            def edge_body(grp, w8i):
                ev16 = evb[pl.ds(grp * 16, 16)]
                for ll in range(16):
                    j = grp * 16 + ll
                    esp = lax.broadcast_in_dim(ev16[ll], (16,), ())
                    for cg in range(4):
                        u = (Pbuf[j, pl.ds(cg * 16, 16)]
                             + Qbuf[j, pl.ds(cg * 16, 16)] + esp * w8i[cg])
                        sig = 1.0 / (1.0 + jnp.exp(-u))
                        v = (Pbuf[j, pl.ds(C + cg * 16, 16)]
                             + Qbuf[j, pl.ds(C + cg * 16, 16)]
                             + esp * w8i[4 + cg])
                        t = jnp.exp(-jnp.abs(v))
                        p = LOG1P_C[0]
                        for cc in LOG1P_C[1:]:
                            p = p * t + cc
                        sp = jnp.maximum(v, 0.0) + p
                        msgb[j, pl.ds(cg * 16, 16)] = sig * sp
                return w8i

            pltpu.sync_copy(msgb, msg_h.at[pl.ds(off, CHUNK)])
            return w8c

        lax.fori_loop(0, NCH_W, chunk_body, w8)

    return k(Pp, Qp, srcp, dstgp, ep, wev)


def _sc_scatter(msg, dstsp, zrows):
    mesh = plsc.VectorSubcoreMesh(core_axis_name="c", subcore_axis_name="s")

    @functools.partial(
        pl.kernel, mesh=mesh,
        out_type=jax.ShapeDtypeStruct((N_PAD, C), jnp.float32),
        scratch_types=[
            pltpu.VMEM((CHUNK,), jnp.int32),
            pltpu.VMEM((CHUNK,), jnp.int32),
            pltpu.VMEM((CHUNK, C), jnp.float32),
            pltpu.VMEM_SHARED((R + 8, C), jnp.float32),
        ])
    def k(msg_h, dsts_h, z_h, agg_h, dstv, lidx, msgb, aggsh):
        cid = lax.axis_index("c")
        sid = lax.axis_index("s")
        for rnd in range(KR // NC):
            base = (cid * (KR // NC) + rnd) * R
            pltpu.sync_copy(z_h.at[pl.ds(0, RT)],
                            aggsh.at[pl.ds(sid * RT, RT)])

            @pl.when(sid == 0)
            def _():
                pltpu.sync_copy(z_h.at[pl.ds(0, 8)], aggsh.at[pl.ds(R, 8)])

            plsc.subcore_barrier()

            def chunk_body(ch, carry):
                off = (sid * NCH_T + ch) * CHUNK
                pltpu.sync_copy(dsts_h.at[pl.ds(off, CHUNK)], dstv)
                pltpu.sync_copy(msg_h.at[pl.ds(off, CHUNK)], msgb)
                for g in range(8):
                    d = dstv[pl.ds(g * 16, 16)]
                    inr = (d >= base) & (d < base + R)
                    lidx[pl.ds(g * 16, 16)] = jnp.where(inr, d - base, R)
                pltpu.sync_copy(msgb, aggsh.at[lidx], add=True)
                return carry

            lax.fori_loop(0, NCH_T, chunk_body, 0)
            plsc.subcore_barrier()
            pltpu.sync_copy(aggsh.at[pl.ds(sid * RT, RT)],
                            agg_h.at[pl.ds(base + sid * RT, RT)])
            plsc.subcore_barrier()

    return k(msg, dstsp, zrows)


def _bucketize(srcp, dstsp, ep):
    dsts2 = dstsp.reshape(E_PAD, 1)
    counts = _bcount(dsts2).reshape(EGRID, NBUCK)
    blk_excl = jnp.cumsum(counts, axis=0) - counts
    tot = jnp.sum(counts, axis=0)
    bstart = jnp.cumsum(tot) - tot
    off = bstart[None, :] + blk_excl
    slots = _slots(dsts2, off.reshape(EGRID, 1, NBUCK)).reshape(E_PAD)
    base2 = (slots >> 7) * 256 + jnp.bitwise_and(slots, 127)
    edc, ebkt = _sc_permute(dstsp, srcp, ep, base2, base2 + 128, slots)

    bs = bstart.astype(jnp.int32)
    be = (bstart + tot).astype(jnp.int32)
    t = jnp.arange(NW, dtype=jnp.int32)
    rows = []
    for r in range(RND):
        if r < 6:
            b = r * NW + t
        else:
            b = jnp.where(t < NBUCK - 6 * NW, 6 * NW + t, NBUCK)
        bc = jnp.minimum(b, NBUCK - 1)
        sr = jnp.where(b < NBUCK, bs[bc], 0)
        er = jnp.where(b < NBUCK, be[bc], 0)
        rows.append(jnp.stack([b, sr, er], axis=-1))
    bnd3 = jnp.stack(rows, axis=1).reshape(NW * RND, 3)
    bounds = jnp.pad(bnd3, ((0, 0), (0, 13))).reshape(-1)
    return edc, ebkt, bounds


_BISECT_JNP_LAYER = False


def _edge_pass(Pp, Qp, we, edc, ebkt, bounds, zrow):
    Pp = jnp.pad(Pp, ((0, N_PAD - N), (0, 0)))
    Qp = jnp.pad(Qp, ((0, N_PAD - N), (0, 0)))
    agg = _sc_layer(Pp, Qp, edc, ebkt, bounds, zrow, we.reshape(8, 16))
    return agg[:N]


def _prj(Wf, bf, Ws, bs):
    A = jnp.concatenate([Wf[:C], Ws[:C]], axis=1)
    B = jnp.concatenate([Wf[C:2 * C], Ws[C:2 * C]], axis=1)
    Ab = jnp.concatenate([bf, bs]).reshape(1, 2 * C)
    we = jnp.concatenate([Wf[2 * C], Ws[2 * C]]).reshape(1, 2 * C)
    return A, Ab, B, we


def kernel(x, edge_index, edge_attr, batch, W0, b0, g0, be0, Wf1, bf1, Ws1,
           bs1, g1, be1, Wf2, bf2, Ws2, bs2, g2, be2, Wf3, bf3, Ws3, bs3, g3,
           be3, fcW1, fcb1, fcW2, fcb2):
    src = edge_index[0]
    dst = edge_index[1]
    e = edge_attr[:, 0]

    srcp = jnp.pad(src, (0, E_PAD - E))
    dstsp = jnp.pad(dst, (0, E_PAD - E), constant_values=DPAD)
    ep = jnp.pad(e, (0, E_PAD - E))
    edc, ebkt, bounds = _bucketize(srcp, dstsp, ep)
    zrow = jnp.zeros((BW + 8, C), jnp.float32)

    layers = [(Wf1, bf1, Ws1, bs1), (Wf2, bf2, Ws2, bs2), (Wf3, bf3, Ws3, bs3)]
    bn = [(g0, be0), (g1, be1), (g2, be2), (g3, be3)]

    y0 = _lin0(x, W0, b0)
    mu, rstd = _moments(y0)
    A, Ab, B, we = _prj(*layers[0])
    h, P, Q = _update(y0, None, mu, rstd, bn[0][0].reshape(1, C),
                      bn[0][1].reshape(1, C), A, Ab, B, True)

    for i in (1, 2, 3):
        agg = _edge_pass(P, Q, we, edc, ebkt, bounds, zrow)
        mu, rstd = _moments(agg)
        g, be = bn[i]
        if i < 3:
            A, Ab, B, we = _prj(*layers[i])
            h, P, Q = _update(agg, h, mu, rstd, g.reshape(1, C),
                              be.reshape(1, C), A, Ab, B, True)
        else:
            return _pool_head(agg, h, mu, rstd, g.reshape(1, C),
                              be.reshape(1, C), batch, fcW1, fcb1, fcW2, fcb2)


# speedup vs baseline: 1.3849x; 1.0007x over previous
import functools
import jax
import jax.numpy as jnp
from jax import lax
from jax.experimental import pallas as pl
from jax.experimental.pallas import tpu as pltpu
from jax.experimental.pallas import tpu_sc as plsc

N = 100000
E = 1600000
G = 256
C = 64
RB = 2000
GRID = N // RB

NC = 2
NS = 16
NW = NC * NS
N_PAD = 100352
E_PAD = 1601536
EW = E_PAD // NW
CHUNK = 128
NCH_W = EW // CHUNK
DPAD = 100351

LOG1P_C = (0.030102247599621303, -0.1301179302884263, 0.2833023836204729,
           -0.4891557820115131, 0.9990102089269793, 2.2132783999840164e-05)

BW = 512
NBUCK = 196
NBJ = NBUCK + 1
AGG_ROWS = NBUCK * BW
EB = 2048
EGRID = E_PAD // EB
E_PAD3 = E_PAD + 256
RND = 7
NCH_MAX = E_PAD // CHUNK


def _stats_body(y_ref, mu_ref, out_ref):
    i = pl.program_id(0)

    @pl.when(i == 0)
    def _():
        out_ref[...] = jnp.zeros_like(out_ref)

    y = y_ref[...]
    yc = y - mu_ref[...]
    s1 = jnp.sum(y, axis=0, keepdims=True)
    s2 = jnp.sum(yc * yc, axis=0, keepdims=True)
    out_ref[...] += jnp.concatenate([s1, s2], axis=0)


def _stats(y, mu):
    return pl.pallas_call(
        _stats_body,
        grid=(GRID,),
        in_specs=[pl.BlockSpec((RB, C), lambda i: (i, 0)),
                  pl.BlockSpec((1, C), lambda i: (0, 0))],
        out_specs=pl.BlockSpec((2, C), lambda i: (0, 0)),
        out_shape=jax.ShapeDtypeStruct((2, C), jnp.float32),
    )(y, mu)


def _moments(y):
    s = _stats(y, jnp.zeros((1, C), jnp.float32))
    mu = (s[0] / N).reshape(1, C)
    s2 = _stats(y, mu)
    var = s2[1] / N
    rstd = 1.0 / jnp.sqrt(var + 1e-5)
    return mu, rstd.reshape(1, C)


def _lin0_body(x_ref, w_ref, b_ref, y_ref):
    y_ref[...] = jnp.dot(x_ref[...], w_ref[...],
                         preferred_element_type=jnp.float32) + b_ref[...]


def _lin0(x, W0, b0):
    return pl.pallas_call(
        _lin0_body,
        grid=(GRID,),
        in_specs=[
            pl.BlockSpec((RB, 12), lambda i: (i, 0)),
            pl.BlockSpec((12, C), lambda i: (0, 0)),
            pl.BlockSpec((1, C), lambda i: (0, 0)),
        ],
        out_specs=pl.BlockSpec((RB, C), lambda i: (i, 0)),
        out_shape=jax.ShapeDtypeStruct((N, C), jnp.float32),
    )(x, W0, b0.reshape(1, C))


def _update_body(use_relu, has_res, y_ref, res_ref, mu_ref, rstd_ref, g_ref,
                 be_ref, A_ref, Ab_ref, B_ref, h_ref, p_ref, q_ref):
    h = (y_ref[...] - mu_ref[...]) * (g_ref[...] * rstd_ref[...]) + be_ref[...]
    if has_res:
        h = h + res_ref[...]
    if use_relu:
        h = jnp.maximum(h, 0.0)
    h_ref[...] = h
    p_ref[...] = jnp.dot(h, A_ref[...],
                         preferred_element_type=jnp.float32) + Ab_ref[...]
    q_ref[...] = jnp.dot(h, B_ref[...], preferred_element_type=jnp.float32)


def _update(y, res, mu, rstd, g, be, A, Ab, B, use_relu):
    has_res = res is not None
    if not has_res:
        res = jnp.zeros((1, C), jnp.float32)
    body = functools.partial(_update_body, use_relu, has_res)
    res_spec = (pl.BlockSpec((RB, C), lambda i: (i, 0)) if has_res
                else pl.BlockSpec((1, C), lambda i: (0, 0)))
    return pl.pallas_call(
        body,
        grid=(GRID,),
        in_specs=[
            pl.BlockSpec((RB, C), lambda i: (i, 0)),
            res_spec,
            pl.BlockSpec((1, C), lambda i: (0, 0)),
            pl.BlockSpec((1, C), lambda i: (0, 0)),
            pl.BlockSpec((1, C), lambda i: (0, 0)),
            pl.BlockSpec((1, C), lambda i: (0, 0)),
            pl.BlockSpec((C, 2 * C), lambda i: (0, 0)),
            pl.BlockSpec((1, 2 * C), lambda i: (0, 0)),
            pl.BlockSpec((C, 2 * C), lambda i: (0, 0)),
        ],
        out_specs=[
            pl.BlockSpec((RB, C), lambda i: (i, 0)),
            pl.BlockSpec((RB, 2 * C), lambda i: (i, 0)),
            pl.BlockSpec((RB, 2 * C), lambda i: (i, 0)),
        ],
        out_shape=[
            jax.ShapeDtypeStruct((N, C), jnp.float32),
            jax.ShapeDtypeStruct((N, 2 * C), jnp.float32),
            jax.ShapeDtypeStruct((N, 2 * C), jnp.float32),
        ],
    )(y, res, mu, rstd, g, be, A, Ab, B)


def _pool_body(agg_ref, h2_ref, mu_ref, rstd_ref, g_ref, be_ref, batch_ref,
               fcW1_ref, fcb1_ref, fcW2_ref, fcb2_ref, out_ref,
               acc_ref, cnt_ref):
    i = pl.program_id(0)

    @pl.when(i == 0)
    def _():
        acc_ref[...] = jnp.zeros_like(acc_ref)
        cnt_ref[...] = jnp.zeros_like(cnt_ref)

    h3 = (agg_ref[...] - mu_ref[...]) * (g_ref[...] * rstd_ref[...]) \
        + be_ref[...] + h2_ref[...]
    oh = (batch_ref[...] ==
          jax.lax.broadcasted_iota(jnp.int32, (1, G), 1)).astype(jnp.float32)
    acc_ref[...] += jax.lax.dot_general(
        oh, h3, (((0,), (0,)), ((), ())), preferred_element_type=jnp.float32)
    cnt_ref[...] += jnp.sum(oh, axis=0, keepdims=True)

    @pl.when(i == GRID - 1)
    def _():
        pooled = acc_ref[...] / jnp.maximum(cnt_ref[...], 1.0).T
        t = jnp.maximum(
            jnp.dot(pooled, fcW1_ref[...], preferred_element_type=jnp.float32)
            + fcb1_ref[...], 0.0)
        out_ref[...] = jnp.dot(t, fcW2_ref[...],
                               preferred_element_type=jnp.float32) + fcb2_ref[...]


def _pool_head(agg3, h2, mu, rstd, g, be, batch, fcW1, fcb1, fcW2, fcb2):
    return pl.pallas_call(
        _pool_body,
        grid=(GRID,),
        in_specs=[
            pl.BlockSpec((RB, C), lambda i: (i, 0)),
            pl.BlockSpec((RB, C), lambda i: (i, 0)),
            pl.BlockSpec((1, C), lambda i: (0, 0)),
            pl.BlockSpec((1, C), lambda i: (0, 0)),
            pl.BlockSpec((1, C), lambda i: (0, 0)),
            pl.BlockSpec((1, C), lambda i: (0, 0)),
            pl.BlockSpec((RB, 1), lambda i: (i, 0)),
            pl.BlockSpec((C, 32), lambda i: (0, 0)),
            pl.BlockSpec((1, 32), lambda i: (0, 0)),
            pl.BlockSpec((32, 1), lambda i: (0, 0)),
            pl.BlockSpec((1, 1), lambda i: (0, 0)),
        ],
        out_specs=pl.BlockSpec((G, 1), lambda i: (0, 0)),
        out_shape=jax.ShapeDtypeStruct((G, 1), jnp.float32),
        scratch_shapes=[pltpu.VMEM((G, C), jnp.float32),
                        pltpu.VMEM((1, G), jnp.float32)],
    )(agg3, h2, mu, rstd, g, be, batch.reshape(N, 1), fcW1,
      fcb1.reshape(1, 32), fcW2, fcb2.reshape(1, 1))


def _bucket_of(d):
    return jnp.minimum(jax.lax.shift_right_logical(d, 9), NBUCK - 1)


def _bcount_body(d_ref, out_ref):
    b = _bucket_of(d_ref[...])
    oh = (b == jax.lax.broadcasted_iota(jnp.int32, (1, NBUCK), 1))
    out_ref[...] = jnp.sum(oh.astype(jnp.float32), axis=0,
                           keepdims=True).reshape(1, 1, NBUCK)


def _bcount(dsts2):
    return pl.pallas_call(
        _bcount_body,
        grid=(EGRID,),
        in_specs=[pl.BlockSpec((EB, 1), lambda i: (i, 0))],
        out_specs=pl.BlockSpec((1, 1, NBUCK), lambda i: (i, 0, 0)),
        out_shape=jax.ShapeDtypeStruct((EGRID, 1, NBUCK), jnp.float32),
    )(dsts2)


def _slots_body(d_ref, off_ref, slot_ref):
    b = _bucket_of(d_ref[...])
    oh = (b == jax.lax.broadcasted_iota(jnp.int32, (1, NBUCK), 1)
          ).astype(jnp.float32)
    oh3 = oh.reshape(EB // 8, 8, NBUCK)
    run = jnp.zeros((EB // 8, NBUCK), jnp.float32)
    parts = []
    for r in range(8):
        parts.append(run[:, None, :])
        run = run + oh3[:, r, :]
    excl8 = jnp.concatenate(parts, axis=1)
    inc = run
    sh = 1
    while sh < EB // 8:
        inc = inc + jnp.concatenate(
            [jnp.zeros((sh, NBUCK), jnp.float32), inc[:-sh]], axis=0)
        sh *= 2
    rank3 = excl8 + (inc - run)[:, None, :]
    rank = jnp.sum((rank3.reshape(EB, NBUCK)
                    + off_ref[...].reshape(1, NBUCK)) * oh, axis=1)
    slot_ref[...] = rank.astype(jnp.int32)[:, None]


def _slots(dsts2, off):
    return pl.pallas_call(
        _slots_body,
        grid=(EGRID,),
        in_specs=[pl.BlockSpec((EB, 1), lambda i: (i, 0)),
                  pl.BlockSpec((1, 1, NBUCK), lambda i: (i, 0, 0))],
        out_specs=pl.BlockSpec((EB, 1), lambda i: (i, 0)),
        out_shape=jax.ShapeDtypeStruct((E_PAD, 1), jnp.int32),
    )(dsts2, off)


def _sc_permute(dstsp, srcp, ep, s0, s1, s2):
    mesh = plsc.VectorSubcoreMesh(core_axis_name="c", subcore_axis_name="s")

    @functools.partial(
        pl.kernel, mesh=mesh,
        out_type=[jax.ShapeDtypeStruct((2 * E_PAD3,), jnp.int32),
                  jax.ShapeDtypeStruct((E_PAD3,), jnp.float32)],
        scratch_types=[
            pltpu.VMEM((3, CHUNK), jnp.int32),
            pltpu.VMEM((CHUNK,), jnp.int32),
            pltpu.VMEM((CHUNK,), jnp.int32),
            pltpu.VMEM((CHUNK,), jnp.float32),
            pltpu.VMEM((256,), jnp.int32),
            pltpu.VMEM((256,), jnp.float32),
            pltpu.SemaphoreType.DMA,
            pltpu.SemaphoreType.DMA,
            pltpu.SemaphoreType.DMA,
        ])
    def k(dsts_h, srcp_h, ep_h, s0_h, s1_h, s2_h, edc_h, eb_h,
          sltv, dv, sv, evb, tl_i, tl_f, sem1, sem2, sem3):
        wid = lax.axis_index("s") * NC + lax.axis_index("c")

        def chunk_body(ch, carry):
            off = wid * EW + ch * CHUNK
            pltpu.sync_copy(s0_h.at[pl.ds(off, CHUNK)], sltv.at[0])
            pltpu.sync_copy(s1_h.at[pl.ds(off, CHUNK)], sltv.at[1])
            pltpu.sync_copy(s2_h.at[pl.ds(off, CHUNK)], sltv.at[2])
            pltpu.sync_copy(dsts_h.at[pl.ds(off, CHUNK)], dv)
            pltpu.sync_copy(srcp_h.at[pl.ds(off, CHUNK)], sv)
            pltpu.sync_copy(ep_h.at[pl.ds(off, CHUNK)], evb)
            c1 = pltpu.async_copy(dv, edc_h.at[sltv.at[0]], sem1)
            c2 = pltpu.async_copy(sv, edc_h.at[sltv.at[1]], sem2)
            c3 = pltpu.async_copy(evb, eb_h.at[sltv.at[2]], sem3)
            c1.wait()
            c2.wait()
            c3.wait()
            return carry

        lax.fori_loop(0, NCH_W, chunk_body, 0)

        @pl.when(wid == 0)
        def _():
            for i in range(16):
                val = DPAD if i < 8 else 0
                tl_i[pl.ds(i * 16, 16)] = jnp.full((16,), val, jnp.int32)
                tl_f[pl.ds(i * 16, 16)] = jnp.zeros((16,), jnp.float32)
            pltpu.sync_copy(tl_i, edc_h.at[pl.ds(2 * E_PAD, 256)])
            pltpu.sync_copy(tl_i, edc_h.at[pl.ds(2 * E_PAD + 256, 256)])
            pltpu.sync_copy(tl_f, eb_h.at[pl.ds(E_PAD, 256)])

    return k(dstsp, srcp, ep, s0, s1, s2)


def _sc_layer(Pp, Qp, edc, ebkt, bounds, zrow, wev):
    mesh = plsc.VectorSubcoreMesh(core_axis_name="c", subcore_axis_name="s")

    @functools.partial(
        pl.kernel, mesh=mesh,
        out_type=jax.ShapeDtypeStruct((AGG_ROWS, C), jnp.float32),
        scratch_types=[
            pltpu.VMEM((RND * 16,), jnp.int32),
            pltpu.VMEM((2 * CHUNK,), jnp.int32),
            pltpu.VMEM((CHUNK,), jnp.float32),
            pltpu.VMEM((CHUNK,), jnp.int32),
            pltpu.VMEM((CHUNK, 2 * C), jnp.float32),
            pltpu.VMEM((CHUNK, 2 * C), jnp.float32),
            pltpu.VMEM((CHUNK, C), jnp.float32),
            pltpu.VMEM((BW + 8, C), jnp.float32),
            pltpu.VMEM((8, 16), jnp.float32),
            pltpu.SemaphoreType.DMA,
            pltpu.SemaphoreType.DMA,
        ])
    def k(Pp_h, Qp_h, edc_h, eb_h, bounds_h, z_h, wev_h, agg_h,
          bnd, ebuf, evb, lidx, Pbuf, Qbuf, msgb, aggt, wevv,
          sem1, sem2):
        tid = lax.axis_index("s") * NC + lax.axis_index("c")
        pltpu.sync_copy(wev_h, wevv)
        pltpu.sync_copy(bounds_h.at[pl.ds(tid * (RND * 16), RND * 16)], bnd)
        w8 = tuple(wevv[i] for i in range(8))

        def round_body(rnd, w8r):
            bv = bnd[pl.ds(rnd * 16, 16)]
            bkt = bv[0]
            s = bv[1]
            e_ = bv[2]
            bbase = bkt * BW
            c0 = pl.multiple_of(jnp.bitwise_and(s, jnp.int32(-128)), 128)
            nch = jax.lax.shift_right_logical(e_ - c0 + 127, 7)

            @pl.when(bkt < NBUCK)
            def _():
                pltpu.sync_copy(z_h, aggt)

            def chunk_body(ch, w8c):
                @pl.when(ch < nch)
                def _():
                    off = c0 + ch * CHUNK
                    boff = pl.multiple_of(off * 2, 128)
                    pltpu.sync_copy(edc_h.at[pl.ds(boff, 2 * CHUNK)], ebuf)
                    pltpu.sync_copy(eb_h.at[pl.ds(off, CHUNK)], evb)
                    for g in range(8):
                        d = ebuf[pl.ds(g * 16, 16)]
                        inb = (d >= bbase) & (d < bbase + BW)
                        lidx[pl.ds(g * 16, 16)] = jnp.where(inb, d - bbase, BW)
                    cp1 = pltpu.async_copy(Pp_h.at[ebuf.at[pl.ds(0, CHUNK)]],
                                           Pbuf, sem1)
                    cp2 = pltpu.async_copy(
                        Qp_h.at[ebuf.at[pl.ds(CHUNK, CHUNK)]], Qbuf, sem2)
                    cp1.wait()
                    cp2.wait()

                    @plsc.parallel_loop(0, CHUNK // 16, carry=w8c)
                    def edge_body(grp, w8i):
                        ev16 = evb[pl.ds(grp * 16, 16)]
                        for ll in range(16):
                            j = grp * 16 + ll
                            esp = lax.broadcast_in_dim(ev16[ll], (16,), ())
                            for cg in range(4):
                                u = (Pbuf[j, pl.ds(cg * 16, 16)]
                                     + Qbuf[j, pl.ds(cg * 16, 16)]
                                     + esp * w8i[cg])
                                sig = 1.0 / (1.0 + jnp.exp(-u))
                                v = (Pbuf[j, pl.ds(C + cg * 16, 16)]
                                     + Qbuf[j, pl.ds(C + cg * 16, 16)]
                                     + esp * w8i[4 + cg])
                                t = jnp.exp(-jnp.abs(v))
                                p = LOG1P_C[0]
                                for cc in LOG1P_C[1:]:
                                    p = p * t + cc
                                sp = jnp.maximum(v, 0.0) + p
                                msgb[j, pl.ds(cg * 16, 16)] = sig * sp
                        return w8i

                    def acc_body(grp, cacc):
                        l16 = lidx[pl.ds(grp * 16, 16)]
                        for ll in range(16):
                            j = grp * 16 + ll
                            lr = l16[ll]
                            for cg in range(4):
                                plsc.addupdate(
                                    aggt.at[lr, pl.ds(cg * 16, 16)],
                                    msgb[j, pl.ds(cg * 16, 16)])
                        return cacc

                    lax.fori_loop(0, CHUNK // 16, acc_body, 0)

                return w8c

            lax.fori_loop(0, NCH_MAX, chunk_body, w8r)

            @pl.when(bkt < NBUCK)
            def _():
                pltpu.sync_copy(aggt.at[pl.ds(0, BW)],
                                agg_h.at[pl.ds(bbase, BW)])

            return w8r

        lax.fori_loop(0, RND, round_body, w8)

    return k(Pp, Qp, edc, ebkt, bounds, zrow, wev)


def _bucketize(srcp, dstsp, ep):
    dsts2 = dstsp.reshape(E_PAD, 1)
    counts = _bcount(dsts2).reshape(EGRID, NBUCK)
    blk_excl = jnp.cumsum(counts, axis=0) - counts
    tot = jnp.sum(counts, axis=0)
    bstart = jnp.cumsum(tot) - tot
    off = bstart[None, :] + blk_excl
    slots = _slots(dsts2, off.reshape(EGRID, 1, NBUCK)).reshape(E_PAD)
    base2 = (slots >> 7) * 256 + jnp.bitwise_and(slots, 127)
    edc, ebkt = _sc_permute(dstsp, srcp, ep, base2, base2 + 128, slots)

    bs = bstart.astype(jnp.int32)
    be = (bstart + tot).astype(jnp.int32)
    t = jnp.arange(NW, dtype=jnp.int32)
    rows = []
    for r in range(RND):
        if r < 6:
            b = r * NW + t
        else:
            b = jnp.where(t < NBUCK - 6 * NW, 6 * NW + t, NBUCK)
        bc = jnp.minimum(b, NBUCK - 1)
        sr = jnp.where(b < NBUCK, bs[bc], 0)
        er = jnp.where(b < NBUCK, be[bc], 0)
        rows.append(jnp.stack([b, sr, er], axis=-1))
    bnd3 = jnp.stack(rows, axis=1).reshape(NW * RND, 3)
    bounds = jnp.pad(bnd3, ((0, 0), (0, 13))).reshape(-1)
    return edc, ebkt, bounds


def _edge_pass(Pp, Qp, we, edc, ebkt, bounds, zrow):
    Pp = jnp.pad(Pp, ((0, N_PAD - N), (0, 0)))
    Qp = jnp.pad(Qp, ((0, N_PAD - N), (0, 0)))
    agg = _sc_layer(Pp, Qp, edc, ebkt, bounds, zrow, we.reshape(8, 16))
    return agg[:N]


def _prj(Wf, bf, Ws, bs):
    A = jnp.concatenate([Wf[:C], Ws[:C]], axis=1)
    B = jnp.concatenate([Wf[C:2 * C], Ws[C:2 * C]], axis=1)
    Ab = jnp.concatenate([bf, bs]).reshape(1, 2 * C)
    we = jnp.concatenate([Wf[2 * C], Ws[2 * C]]).reshape(1, 2 * C)
    return A, Ab, B, we


def kernel(x, edge_index, edge_attr, batch, W0, b0, g0, be0, Wf1, bf1, Ws1,
           bs1, g1, be1, Wf2, bf2, Ws2, bs2, g2, be2, Wf3, bf3, Ws3, bs3, g3,
           be3, fcW1, fcb1, fcW2, fcb2):
    src = edge_index[0]
    dst = edge_index[1]
    e = edge_attr[:, 0]

    srcp = jnp.pad(src, (0, E_PAD - E))
    dstsp = jnp.pad(dst, (0, E_PAD - E), constant_values=DPAD)
    ep = jnp.pad(e, (0, E_PAD - E))
    edc, ebkt, bounds = _bucketize(srcp, dstsp, ep)
    zrow = jnp.zeros((BW + 8, C), jnp.float32)

    layers = [(Wf1, bf1, Ws1, bs1), (Wf2, bf2, Ws2, bs2), (Wf3, bf3, Ws3, bs3)]
    bn = [(g0, be0), (g1, be1), (g2, be2), (g3, be3)]

    y0 = _lin0(x, W0, b0)
    mu, rstd = _moments(y0)
    A, Ab, B, we = _prj(*layers[0])
    h, P, Q = _update(y0, None, mu, rstd, bn[0][0].reshape(1, C),
                      bn[0][1].reshape(1, C), A, Ab, B, True)

    for i in (1, 2, 3):
        agg = _edge_pass(P, Q, we, edc, ebkt, bounds, zrow)
        mu, rstd = _moments(agg)
        g, be = bn[i]
        if i < 3:
            A, Ab, B, we = _prj(*layers[i])
            h, P, Q = _update(agg, h, mu, rstd, g.reshape(1, C),
                              be.reshape(1, C), A, Ab, B, True)
        else:
            return _pool_head(agg, h, mu, rstd, g.reshape(1, C),
                              be.reshape(1, C), batch, fcW1, fcb1, fcW2, fcb2)
